# baseline (speedup 1.0000x reference)
import jax
import jax.numpy as jnp
from jax.experimental import pallas as pl
from jax.experimental.pallas import tpu as pltpu

_B, _L, _N, _D, _F, _E, _K, _P, _LAYERS = 32, 96, 32, 16, 64, 4, 2, 96, 2
_EF = _E * _F
_BLK = 8
_G = _B // _BLK
_T = _BLK * _L * _N
_TB = _L * _N
_S = 2 * _LAYERS * _E


def _top2(lgT):
    f32 = jnp.float32

    def first_max_onehot(x):
        v = jnp.max(x, axis=0, keepdims=True)
        rows = []
        seen = jnp.zeros_like(v)
        for e in range(_E):
            eq = (x[e:e + 1] == v).astype(f32)
            rows.append(eq * (1.0 - seen))
            seen = jnp.maximum(seen, eq)
        return jnp.concatenate(rows, axis=0), v

    oh1, v1 = first_max_onehot(lgT)
    masked = jnp.where(oh1 > 0, -jnp.inf, lgT)
    oh2, v2 = first_max_onehot(masked)
    e2 = jnp.exp(v2 - v1)
    den = 1.0 + e2
    return oh1 * (1.0 / den) + oh2 * (e2 / den)


def _model_kernel(x_ref, startW_ref, startb_ref, wg_ref, W1_ref, b1_ref,
                  W2_ref, b2_ref, projW_ref, projb_ref,
                  dec_ref, stats_ref, M_ref):
    f32 = jnp.float32

    xt = x_ref[0]
    outT = startW_ref[:] * xt + startb_ref[:]

    stats = []
    for l in range(_LAYERS):
        lgT = jnp.dot(wg_ref[l], outT, preferred_element_type=f32)
        gatesT = _top2(lgT)
        stats.append(jnp.sum(gatesT, axis=1, keepdims=True))
        stats.append(jnp.sum((gatesT > 0).astype(f32), axis=1, keepdims=True))

        hT = jax.nn.gelu(jnp.dot(W1_ref[l], outT, preferred_element_type=f32)
                         + b1_ref[l])
        yT = None
        for e in range(_E):
            oeT = (jnp.dot(W2_ref[l][:, e * _F:(e + 1) * _F],
                           hT[e * _F:(e + 1) * _F],
                           preferred_element_type=f32)
                   + b2_ref[l][:, e:e + 1])
            term = gatesT[e:e + 1] * oeT
            yT = term if yT is None else yT + term
        outT = outT + yT

    stats_ref[0] = jnp.concatenate(stats, axis=0)

    for k in range(_BLK):
        for l in range(_L):
            M_ref[k, l * _D:(l + 1) * _D, :] = (
                outT[:, k * _TB + l * _N:k * _TB + (l + 1) * _N])
    for k in range(_BLK):
        dec_ref[k] = (jnp.dot(projW_ref[:], M_ref[k], preferred_element_type=f32)
                      + projb_ref[:])


def _balance_kernel(stats_ref, bal_ref):
    f32 = jnp.float32
    s = jnp.sum(stats_ref[:], axis=0)
    bal = jnp.zeros((1, 1), dtype=f32)
    for i in range(2 * _LAYERS):
        v = s[i * _E:(i + 1) * _E]
        m = jnp.sum(v, keepdims=True) / _E
        var = jnp.sum((v - m) ** 2, keepdims=True) / (_E - 1)
        bal = bal + var / (m * m + 1e-10)
    bal_ref[:] = bal


def kernel(x_enc, x_mark_enc, x_dec, x_mark_dec, start_W, start_b, w_gate,
           W1, b1, W2, b2, proj_W, proj_b):
    f32 = jnp.float32
    wgT = jnp.transpose(w_gate, (0, 2, 1))
    W1T = jnp.transpose(W1, (0, 1, 3, 2)).reshape(_LAYERS, _EF, _D)
    b1T = b1.reshape(_LAYERS, _EF, 1)
    W2T = jnp.transpose(W2, (0, 3, 1, 2)).reshape(_LAYERS, _D, _EF)
    b2T = jnp.transpose(b2, (0, 2, 1))
    startWT = start_W.reshape(_D, 1)
    startbT = start_b.reshape(_D, 1)
    projWT = jnp.transpose(proj_W, (1, 0))
    projbT = proj_b.reshape(_P, 1)
    xp = x_enc.reshape(_G, 1, _T)

    full = lambda shape: pl.BlockSpec(shape, lambda b: (0,) * len(shape))
    dec, stats = pl.pallas_call(
        _model_kernel,
        grid=(_G,),
        in_specs=[
            pl.BlockSpec((1, 1, _T), lambda b: (b, 0, 0)),
            full((_D, 1)),
            full((_D, 1)),
            full((_LAYERS, _E, _D)),
            full((_LAYERS, _EF, _D)),
            full((_LAYERS, _EF, 1)),
            full((_LAYERS, _D, _EF)),
            full((_LAYERS, _D, _E)),
            full((_P, _L * _D)),
            full((_P, 1)),
        ],
        out_specs=[
            pl.BlockSpec((_BLK, _P, _N), lambda b: (b, 0, 0)),
            pl.BlockSpec((1, _S, 1), lambda b: (b, 0, 0)),
        ],
        out_shape=[
            jax.ShapeDtypeStruct((_B, _P, _N), f32),
            jax.ShapeDtypeStruct((_G, _S, 1), f32),
        ],
        scratch_shapes=[
            pltpu.VMEM((_BLK, _L * _D, _N), f32),
        ],
        compiler_params=pltpu.CompilerParams(
            dimension_semantics=("parallel",),
        ),
    )(xp, startWT, startbT, wgT, W1T, b1T, W2T, b2T, projWT, projbT)

    bal = pl.pallas_call(
        _balance_kernel,
        out_shape=jax.ShapeDtypeStruct((1, 1), f32),
    )(stats)
    return dec, bal[0, 0]
